# baseline (speedup 1.0000x reference)
import jax
import jax.numpy as jnp
from jax import lax
from jax.experimental import pallas as pl
from jax.experimental.pallas import tpu as pltpu

N_V = 1000
N_D = 64
BATCH = 4096
HIST = 200
B_TOTAL = BATCH * HIST
VPAD = 1024
RBLK = 1024
NBLK = B_TOTAL // RBLK


def _tc_body(idx_ref, w_ref, out_ref):
  idxv = idx_ref[0]
  cols = lax.broadcasted_iota(jnp.int32, (RBLK, VPAD), 1)
  oh = (idxv == cols).astype(jnp.bfloat16)
  out_ref[...] = jnp.dot(oh, w_ref[...],
                         preferred_element_type=jnp.float32)


@jax.jit
def kernel(input_, W):
  idx3 = input_.reshape(NBLK, RBLK, 1)
  wp = jnp.zeros((VPAD, N_D), jnp.bfloat16).at[:N_V].set(
      W.astype(jnp.bfloat16))
  out = pl.pallas_call(
      _tc_body,
      grid=(NBLK,),
      in_specs=[
          pl.BlockSpec((1, RBLK, 1), lambda i: (i, 0, 0)),
          pl.BlockSpec((VPAD, N_D), lambda i: (0, 0)),
      ],
      out_specs=pl.BlockSpec((RBLK, N_D), lambda i: (i, 0)),
      out_shape=jax.ShapeDtypeStruct((B_TOTAL, N_D), jnp.float32),
  )(idx3, wp)
  return out.reshape(BATCH, HIST, N_D)
